# baseline (speedup 1.0000x reference)
import functools

import jax
import jax.numpy as jnp
from jax import lax
from jax.experimental import pallas as pl
from jax.experimental.pallas import tpu as pltpu
from jax.experimental.pallas import tpu_sc as plsc

_N = 10000
_E = 320000
_D = 128
_LN_EPS = 1e-5

_NPAD = 10240
_B = 128
_CH = 80
_NW = 32
_EPAD = _NW * _CH * _B
_REAL_CHUNKS = _E // _B
_RPT = _NPAD // 16
_NB = pl.cdiv(_N, _B)
_HCOL = _NPAD // _B


def _deg_body(ep_hbm, out_hbm, e_pk, t_src, t_dst, upd, zbuf,
              dego_sh, degi_sh, semo, semi):
    c = lax.axis_index("c")
    s = lax.axis_index("s")
    w = c * 16 + s
    for k in range(_RPT // 16):
        zbuf[pl.ds(k * 16, 16)] = jnp.zeros((16,), jnp.float32)
    for k in range(_B // 16):
        upd[pl.ds(k * 16, 16)] = jnp.ones((16,), jnp.float32)
    pltpu.sync_copy(zbuf, dego_sh.at[pl.ds(s * _RPT, _RPT)])
    pltpu.sync_copy(zbuf, degi_sh.at[pl.ds(s * _RPT, _RPT)])
    pltpu.sync_copy(ep_hbm.at[w], e_pk)
    plsc.subcore_barrier()

    def chunk(j, carry):
        valid = w * _CH + j < _REAL_CHUNKS
        for k in range(_B // 16):
            v = e_pk[j, pl.ds(k * 16, 16)]
            sv = v & 16383
            dv = lax.shift_right_logical(v, 14)
            trash = (_NPAD + k * 16
                     + lax.broadcasted_iota(jnp.int32, (16,), 0))
            ts = (sv & 127) * _HCOL + lax.shift_right_logical(sv, 7)
            td = (dv & 127) * _HCOL + lax.shift_right_logical(dv, 7)
            t_src[j, pl.ds(k * 16, 16)] = jnp.where(valid, ts, trash)
            t_dst[j, pl.ds(k * 16, 16)] = jnp.where(valid, td, trash)
        pltpu.async_copy(upd, dego_sh.at[t_src.at[j]], semo, add=True)
        pltpu.async_copy(upd, degi_sh.at[t_dst.at[j]], semi, add=True)
        return carry

    lax.fori_loop(0, _CH, chunk, 0)

    def drain(j, carry):
        pltpu.make_async_copy(upd, dego_sh.at[t_src.at[j]], semo).wait()
        pltpu.make_async_copy(upd, degi_sh.at[t_dst.at[j]], semi).wait()
        return carry

    lax.fori_loop(0, _CH, drain, 0)
    plsc.subcore_barrier()
    pltpu.sync_copy(dego_sh.at[pl.ds(s * _RPT, _RPT)],
                    out_hbm.at[c, 0, pl.ds(s * _RPT, _RPT)])
    pltpu.sync_copy(degi_sh.at[pl.ds(s * _RPT, _RPT)],
                    out_hbm.at[c, 1, pl.ds(s * _RPT, _RPT)])


def _agg_body(xs_hbm, ep_hbm, z_hbm, out0_hbm, out1_hbm,
              e_pk, sidx, didx, eidx, rows0, rows1, agg_sh, sem0, sem1):
    c = lax.axis_index("c")
    s = lax.axis_index("s")
    w = c * 16 + s

    for k in range(_CH // 16):
        eidx[pl.ds(k * 16, 16)] = (
            w * _CH + k * 16 + lax.broadcasted_iota(jnp.int32, (16,), 0))
    pltpu.async_copy(ep_hbm.at[eidx], e_pk, sem0)
    pltpu.async_copy(z_hbm.at[pl.ds(s * _RPT, _RPT)],
                     agg_sh.at[pl.ds(s * _RPT, _RPT)], sem1)
    pltpu.make_async_copy(ep_hbm.at[eidx], e_pk, sem0).wait()
    pltpu.make_async_copy(z_hbm.at[pl.ds(s * _RPT, _RPT)],
                          agg_sh.at[pl.ds(s * _RPT, _RPT)], sem1).wait()
    plsc.subcore_barrier()

    def unpack(j, p):
        for k in range(_B // 16):
            v = e_pk[j, pl.ds(k * 16, 16)]
            sidx[p, pl.ds(k * 16, 16)] = v & 16383
            didx[p, pl.ds(k * 16, 16)] = lax.shift_right_logical(v, 14)

    unpack(0, 0)
    pltpu.async_copy(xs_hbm.at[sidx.at[0]], rows0, sem0)
    unpack(1, 1)

    def step(g, carry):
        j0 = 2 * g
        j1 = j0 + 1
        pltpu.async_copy(xs_hbm.at[sidx.at[1]], rows1, sem1)
        pltpu.make_async_copy(xs_hbm.at[sidx.at[0]], rows0, sem0).wait()
        pltpu.sync_copy(rows0, agg_sh.at[didx.at[0]], add=True)

        @pl.when(g < _CH // 2 - 1)
        def _():
            unpack(j0 + 2, 0)
            pltpu.async_copy(xs_hbm.at[sidx.at[0]], rows0, sem0)

        pltpu.make_async_copy(xs_hbm.at[sidx.at[1]], rows1, sem1).wait()
        pltpu.sync_copy(rows1, agg_sh.at[didx.at[1]], add=True)

        @pl.when(g < _CH // 2 - 1)
        def _():
            unpack(j1 + 2, 1)

        return carry

    lax.fori_loop(0, _CH // 2, step, 0)
    plsc.subcore_barrier()

    @pl.when(c == 0)
    def _():
        pltpu.sync_copy(agg_sh.at[pl.ds(s * _RPT, _RPT)],
                        out0_hbm.at[pl.ds(s * _RPT, _RPT)])

    @pl.when(c == 1)
    def _():
        pltpu.sync_copy(agg_sh.at[pl.ds(s * _RPT, _RPT)],
                        out1_hbm.at[pl.ds(s * _RPT, _RPT)])


@functools.cache
def _sc_kernels():
    mesh = plsc.VectorSubcoreMesh(core_axis_name="c", subcore_axis_name="s")
    deg = pl.kernel(
        _deg_body,
        out_type=jax.ShapeDtypeStruct((2, 2, _NPAD), jnp.float32),
        mesh=mesh,
        scratch_types=[
            pltpu.VMEM((_CH, _B), jnp.int32),
            pltpu.VMEM((_CH, _B), jnp.int32),
            pltpu.VMEM((_CH, _B), jnp.int32),
            pltpu.VMEM((_B,), jnp.float32),
            pltpu.VMEM((_RPT,), jnp.float32),
            pltpu.VMEM_SHARED((_NPAD + _B,), jnp.float32),
            pltpu.VMEM_SHARED((_NPAD + _B,), jnp.float32),
            pltpu.SemaphoreType.DMA,
            pltpu.SemaphoreType.DMA,
        ],
    )
    agg = pl.kernel(
        _agg_body,
        out_type=[jax.ShapeDtypeStruct((_NPAD, _D), jnp.float32),
                  jax.ShapeDtypeStruct((_NPAD, _D), jnp.float32)],
        mesh=mesh,
        scratch_types=[
            pltpu.VMEM((_CH, _B), jnp.int32),
            pltpu.VMEM((2, _B), jnp.int32),
            pltpu.VMEM((2, _B), jnp.int32),
            pltpu.VMEM((_CH,), jnp.int32),
            pltpu.VMEM((_B, _D), jnp.float32),
            pltpu.VMEM((_B, _D), jnp.float32),
            pltpu.VMEM_SHARED((_NPAD, _D), jnp.float32),
            pltpu.SemaphoreType.DMA,
            pltpu.SemaphoreType.DMA,
        ],
    )
    return deg, agg


def _deg_col(d_ref, i):
    d = d_ref[0, 0] + d_ref[1, 0]
    lane = lax.broadcasted_iota(jnp.int32, (_B, _HCOL), 1)
    col = jnp.sum(jnp.where(lane == i, d, 0.0), axis=1, keepdims=True)
    return lax.rsqrt(jnp.maximum(col, 1.0))


def _xs_call(x, degp4):
    def body(x_ref, d_ref, o_ref):
        ro = _deg_col(d_ref, pl.program_id(0))
        o_ref[...] = x_ref[...] * ro

    return pl.pallas_call(
        body,
        grid=(_NB,),
        in_specs=[
            pl.BlockSpec((_B, _D), lambda i: (i, 0)),
            pl.BlockSpec((2, 1, _B, _HCOL), lambda i: (0, 0, 0, 0)),
        ],
        out_specs=pl.BlockSpec((_B, _D), lambda i: (i, 0)),
        out_shape=jax.ShapeDtypeStruct((_N, _D), jnp.float32),
    )(x, degp4)


def _fin_call(aggp0, aggp1, degp4, W, b2, g2, be2, hb2):
    def body(p0_ref, p1_ref, d_ref, w_ref, b_ref, g_ref, be_ref, hb_ref,
             o_ref):
        a = p0_ref[...] + p1_ref[...]
        rin = _deg_col(d_ref, pl.program_id(0))
        t = jnp.dot(a, w_ref[...], preferred_element_type=jnp.float32)
        h = t * rin + b_ref[...]
        mu = jnp.mean(h, axis=1, keepdims=True)
        hc = h - mu
        var = jnp.mean(hc * hc, axis=1, keepdims=True)
        hn = hc * lax.rsqrt(var + _LN_EPS)
        o_ref[...] = hn * g_ref[...] + be_ref[...] + hb_ref[...]

    vec_spec = pl.BlockSpec((1, _D), lambda i: (0, 0))
    return pl.pallas_call(
        body,
        grid=(_NB,),
        in_specs=[
            pl.BlockSpec((_B, _D), lambda i: (i, 0)),
            pl.BlockSpec((_B, _D), lambda i: (i, 0)),
            pl.BlockSpec((2, 1, _B, _HCOL), lambda i: (0, 1, 0, 0)),
            pl.BlockSpec((_D, _D), lambda i: (0, 0)),
            vec_spec, vec_spec, vec_spec, vec_spec,
        ],
        out_specs=pl.BlockSpec((_B, _D), lambda i: (i, 0)),
        out_shape=jax.ShapeDtypeStruct((_N, _D), jnp.float32),
    )(aggp0, aggp1, degp4, W, b2, g2, be2, hb2)


def kernel(x, edge_index, W, b_conv, ln_gamma, ln_beta, h_bias):
    src = edge_index[0]
    dst = edge_index[1]
    npe = _EPAD - _E
    ar = jnp.arange(npe, dtype=jnp.int32)
    pad_src = (ar * 37) % _N
    pad_dst = _N + (ar % (_NPAD - _N))
    src_p = jnp.concatenate([src, pad_src])
    dst_p = jnp.concatenate([dst, pad_dst])
    ep = (dst_p * 16384 + src_p).reshape(_NW, _CH, _B)

    deg_k, agg_k = _sc_kernels()
    deg = deg_k(ep)
    degp4 = deg.reshape(2, 2, _B, _HCOL)
    xs = _xs_call(x, degp4)
    zer = jnp.zeros((_NPAD, _D), jnp.float32)
    aggp0, aggp1 = agg_k(xs, ep.reshape(_NW * _CH, _B), zer)
    b2 = b_conv.reshape(1, _D)
    g2 = ln_gamma.reshape(1, _D)
    be2 = ln_beta.reshape(1, _D)
    hb2 = h_bias.reshape(1, _D)
    return _fin_call(aggp0, aggp1, degp4, W, b2, g2, be2, hb2)


# speedup vs baseline: 1.0254x; 1.0254x over previous
import functools

import jax
import jax.numpy as jnp
from jax import lax
from jax.experimental import pallas as pl
from jax.experimental.pallas import tpu as pltpu
from jax.experimental.pallas import tpu_sc as plsc

_N = 10000
_E = 320000
_D = 128
_LN_EPS = 1e-5

_NPAD = 10240
_B = 128
_CH = 80
_NW = 32
_EPAD = _NW * _CH * _B
_REAL_CHUNKS = _E // _B
_RPT = _NPAD // 16
_NB = pl.cdiv(_N, _B)
_HCOL = _NPAD // _B


def _deg_body(ep_hbm, out_hbm, e_pk, t_src, t_dst, upd, zbuf,
              dego_sh, degi_sh, semo, semi):
    c = lax.axis_index("c")
    s = lax.axis_index("s")
    w = c * 16 + s
    for k in range(_RPT // 16):
        zbuf[pl.ds(k * 16, 16)] = jnp.zeros((16,), jnp.float32)
    for k in range(_B // 16):
        upd[pl.ds(k * 16, 16)] = jnp.ones((16,), jnp.float32)
    pltpu.sync_copy(zbuf, dego_sh.at[pl.ds(s * _RPT, _RPT)])
    pltpu.sync_copy(zbuf, degi_sh.at[pl.ds(s * _RPT, _RPT)])
    pltpu.sync_copy(ep_hbm.at[w], e_pk)
    plsc.subcore_barrier()

    def chunk(j, carry):
        valid = w * _CH + j < _REAL_CHUNKS
        for k in range(_B // 16):
            v = e_pk[j, pl.ds(k * 16, 16)]
            sv = v & 16383
            dv = lax.shift_right_logical(v, 14)
            trash = (_NPAD + k * 16
                     + lax.broadcasted_iota(jnp.int32, (16,), 0))
            ts = (sv & 127) * _HCOL + lax.shift_right_logical(sv, 7)
            td = (dv & 127) * _HCOL + lax.shift_right_logical(dv, 7)
            t_src[j, pl.ds(k * 16, 16)] = jnp.where(valid, ts, trash)
            t_dst[j, pl.ds(k * 16, 16)] = jnp.where(valid, td, trash)
        pltpu.async_copy(upd, dego_sh.at[t_src.at[j]], semo, add=True)
        pltpu.async_copy(upd, degi_sh.at[t_dst.at[j]], semi, add=True)
        return carry

    lax.fori_loop(0, _CH, chunk, 0)

    def drain(j, carry):
        pltpu.make_async_copy(upd, dego_sh.at[t_src.at[j]], semo).wait()
        pltpu.make_async_copy(upd, degi_sh.at[t_dst.at[j]], semi).wait()
        return carry

    lax.fori_loop(0, _CH, drain, 0)
    plsc.subcore_barrier()
    pltpu.sync_copy(dego_sh.at[pl.ds(s * _RPT, _RPT)],
                    out_hbm.at[c, 0, pl.ds(s * _RPT, _RPT)])
    pltpu.sync_copy(degi_sh.at[pl.ds(s * _RPT, _RPT)],
                    out_hbm.at[c, 1, pl.ds(s * _RPT, _RPT)])


def _agg_body(xs_hbm, ep_hbm, out0_hbm, out1_hbm,
              e_pk, sidx, didx, eidx, rows0, rows1, agg_sh, sem0, sem1):
    c = lax.axis_index("c")
    s = lax.axis_index("s")
    w = c * 16 + s

    for k in range(_CH // 16):
        eidx[pl.ds(k * 16, 16)] = (
            w * _CH + k * 16 + lax.broadcasted_iota(jnp.int32, (16,), 0))
    pltpu.async_copy(ep_hbm.at[eidx], e_pk, sem0)

    def zrow(i, carry):
        for k in range(_D // 16):
            rows0[i, pl.ds(k * 16, 16)] = jnp.zeros((16,), jnp.float32)
        return carry

    lax.fori_loop(0, _B, zrow, 0)
    for t in range(_RPT // _B):
        pltpu.sync_copy(rows0, agg_sh.at[pl.ds(s * _RPT + t * _B, _B)])
    pltpu.make_async_copy(ep_hbm.at[eidx], e_pk, sem0).wait()
    plsc.subcore_barrier()

    def unpack(j, p):
        for k in range(_B // 16):
            v = e_pk[j, pl.ds(k * 16, 16)]
            sidx[p, pl.ds(k * 16, 16)] = v & 16383
            didx[p, pl.ds(k * 16, 16)] = lax.shift_right_logical(v, 14)

    unpack(0, 0)
    pltpu.async_copy(xs_hbm.at[sidx.at[0]], rows0, sem0)
    unpack(1, 1)

    def step(g, carry):
        j0 = 2 * g
        j1 = j0 + 1
        pltpu.async_copy(xs_hbm.at[sidx.at[1]], rows1, sem1)
        pltpu.make_async_copy(xs_hbm.at[sidx.at[0]], rows0, sem0).wait()
        pltpu.sync_copy(rows0, agg_sh.at[didx.at[0]], add=True)

        @pl.when(g < _CH // 2 - 1)
        def _():
            unpack(j0 + 2, 0)
            pltpu.async_copy(xs_hbm.at[sidx.at[0]], rows0, sem0)

        pltpu.make_async_copy(xs_hbm.at[sidx.at[1]], rows1, sem1).wait()
        pltpu.sync_copy(rows1, agg_sh.at[didx.at[1]], add=True)

        @pl.when(g < _CH // 2 - 1)
        def _():
            unpack(j1 + 2, 1)

        return carry

    lax.fori_loop(0, _CH // 2, step, 0)
    plsc.subcore_barrier()

    @pl.when(c == 0)
    def _():
        pltpu.sync_copy(agg_sh.at[pl.ds(s * _RPT, _RPT)],
                        out0_hbm.at[pl.ds(s * _RPT, _RPT)])

    @pl.when(c == 1)
    def _():
        pltpu.sync_copy(agg_sh.at[pl.ds(s * _RPT, _RPT)],
                        out1_hbm.at[pl.ds(s * _RPT, _RPT)])


@functools.cache
def _sc_kernels():
    mesh = plsc.VectorSubcoreMesh(core_axis_name="c", subcore_axis_name="s")
    deg = pl.kernel(
        _deg_body,
        out_type=jax.ShapeDtypeStruct((2, 2, _NPAD), jnp.float32),
        mesh=mesh,
        scratch_types=[
            pltpu.VMEM((_CH, _B), jnp.int32),
            pltpu.VMEM((_CH, _B), jnp.int32),
            pltpu.VMEM((_CH, _B), jnp.int32),
            pltpu.VMEM((_B,), jnp.float32),
            pltpu.VMEM((_RPT,), jnp.float32),
            pltpu.VMEM_SHARED((_NPAD + _B,), jnp.float32),
            pltpu.VMEM_SHARED((_NPAD + _B,), jnp.float32),
            pltpu.SemaphoreType.DMA,
            pltpu.SemaphoreType.DMA,
        ],
    )
    agg = pl.kernel(
        _agg_body,
        out_type=[jax.ShapeDtypeStruct((_NPAD, _D), jnp.float32),
                  jax.ShapeDtypeStruct((_NPAD, _D), jnp.float32)],
        mesh=mesh,
        scratch_types=[
            pltpu.VMEM((_CH, _B), jnp.int32),
            pltpu.VMEM((2, _B), jnp.int32),
            pltpu.VMEM((2, _B), jnp.int32),
            pltpu.VMEM((_CH,), jnp.int32),
            pltpu.VMEM((_B, _D), jnp.float32),
            pltpu.VMEM((_B, _D), jnp.float32),
            pltpu.VMEM_SHARED((_NPAD, _D), jnp.float32),
            pltpu.SemaphoreType.DMA,
            pltpu.SemaphoreType.DMA,
        ],
    )
    return deg, agg


def _deg_col(d_ref, i):
    d = d_ref[0, 0] + d_ref[1, 0]
    lane = lax.broadcasted_iota(jnp.int32, (_B, _HCOL), 1)
    col = jnp.sum(jnp.where(lane == i, d, 0.0), axis=1, keepdims=True)
    return lax.rsqrt(jnp.maximum(col, 1.0))


def _xs_call(x, degp4):
    def body(x_ref, d_ref, o_ref):
        ro = _deg_col(d_ref, pl.program_id(0))
        o_ref[...] = x_ref[...] * ro

    return pl.pallas_call(
        body,
        grid=(_NB,),
        in_specs=[
            pl.BlockSpec((_B, _D), lambda i: (i, 0)),
            pl.BlockSpec((2, 1, _B, _HCOL), lambda i: (0, 0, 0, 0)),
        ],
        out_specs=pl.BlockSpec((_B, _D), lambda i: (i, 0)),
        out_shape=jax.ShapeDtypeStruct((_N, _D), jnp.float32),
    )(x, degp4)


def _fin_call(aggp0, aggp1, degp4, W, b2, g2, be2, hb2):
    def body(p0_ref, p1_ref, d_ref, w_ref, b_ref, g_ref, be_ref, hb_ref,
             o_ref):
        a = p0_ref[...] + p1_ref[...]
        rin = _deg_col(d_ref, pl.program_id(0))
        t = jnp.dot(a, w_ref[...], preferred_element_type=jnp.float32)
        h = t * rin + b_ref[...]
        mu = jnp.mean(h, axis=1, keepdims=True)
        hc = h - mu
        var = jnp.mean(hc * hc, axis=1, keepdims=True)
        hn = hc * lax.rsqrt(var + _LN_EPS)
        o_ref[...] = hn * g_ref[...] + be_ref[...] + hb_ref[...]

    vec_spec = pl.BlockSpec((1, _D), lambda i: (0, 0))
    return pl.pallas_call(
        body,
        grid=(_NB,),
        in_specs=[
            pl.BlockSpec((_B, _D), lambda i: (i, 0)),
            pl.BlockSpec((_B, _D), lambda i: (i, 0)),
            pl.BlockSpec((2, 1, _B, _HCOL), lambda i: (0, 1, 0, 0)),
            pl.BlockSpec((_D, _D), lambda i: (0, 0)),
            vec_spec, vec_spec, vec_spec, vec_spec,
        ],
        out_specs=pl.BlockSpec((_B, _D), lambda i: (i, 0)),
        out_shape=jax.ShapeDtypeStruct((_N, _D), jnp.float32),
    )(aggp0, aggp1, degp4, W, b2, g2, be2, hb2)


def kernel(x, edge_index, W, b_conv, ln_gamma, ln_beta, h_bias):
    src = edge_index[0]
    dst = edge_index[1]
    npe = _EPAD - _E
    ar = jnp.arange(npe, dtype=jnp.int32)
    pad_src = (ar * 37) % _N
    pad_dst = _N + (ar % (_NPAD - _N))
    src_p = jnp.concatenate([src, pad_src])
    dst_p = jnp.concatenate([dst, pad_dst])
    ep = (dst_p * 16384 + src_p).reshape(_NW, _CH, _B)

    deg_k, agg_k = _sc_kernels()
    deg = deg_k(ep)
    degp4 = deg.reshape(2, 2, _B, _HCOL)
    xs = _xs_call(x, degp4)
    aggp0, aggp1 = agg_k(xs, ep.reshape(_NW * _CH, _B))
    b2 = b_conv.reshape(1, _D)
    g2 = ln_gamma.reshape(1, _D)
    be2 = ln_beta.reshape(1, _D)
    hb2 = h_bias.reshape(1, _D)
    return _fin_call(aggp0, aggp1, degp4, W, b2, g2, be2, hb2)
